# baseline (speedup 1.0000x reference)
import functools

import jax
import jax.numpy as jnp
from jax import lax
from jax.experimental import pallas as pl
from jax.experimental.pallas import tpu as pltpu
from jax.experimental.pallas import tpu_sc as plsc

NC = 2
NS = 16
NW = NC * NS
LANES = 16


def _sc_pool_sum(x, table):
    B, L = x.shape
    V, D = table.shape
    assert B % NW == 0 and D % LANES == 0
    rpw = B // NW
    nch = D // LANES

    mesh = plsc.VectorSubcoreMesh(core_axis_name="c", subcore_axis_name="s")

    @functools.partial(
        pl.kernel,
        out_type=jax.ShapeDtypeStruct((B, D), jnp.float32),
        mesh=mesh,
        scratch_types=[
            pltpu.VMEM((rpw, L), jnp.int32),
            pltpu.VMEM((L, D), jnp.float32),
            pltpu.VMEM((L, D), jnp.float32),
            pltpu.VMEM((rpw, D), jnp.float32),
            pltpu.SemaphoreType.DMA,
            pltpu.SemaphoreType.DMA,
        ],
    )
    def pool(x_hbm, table_hbm, out_hbm, idx_v, rows0, rows1, out_v, sem0, sem1):
        wid = lax.axis_index("s") * NC + lax.axis_index("c")
        base = wid * rpw
        pltpu.sync_copy(x_hbm.at[pl.ds(base, rpw)], idx_v)

        bufs = (rows0, rows1)
        sems = (sem0, sem1)

        pltpu.async_copy(table_hbm.at[idx_v.at[0]], bufs[0], sems[0])

        def body(i, carry):
            for b in range(2):
                r = i * 2 + b
                buf = bufs[b]
                pltpu.make_async_copy(table_hbm.at[idx_v.at[0]], buf,
                                      sems[b]).wait()

                nxt = r + 1

                @pl.when(nxt < rpw)
                def _():
                    pltpu.async_copy(table_hbm.at[idx_v.at[nxt]],
                                     bufs[(b + 1) % 2], sems[(b + 1) % 2])

                for c in range(nch):
                    ch = pl.ds(c * LANES, LANES)
                    acc = buf[0, ch] + buf[1, ch]
                    for j in range(2, L - 1, 2):
                        acc = acc + (buf[j, ch] + buf[j + 1, ch])
                    if L % 2:
                        acc = acc + buf[L - 1, ch]
                    out_v[r, ch] = acc
            return carry

        lax.fori_loop(0, rpw // 2, body, 0)
        pltpu.sync_copy(out_v, out_hbm.at[pl.ds(base, rpw)])

    return pool(x, table)


def _mlp_body(pad_idx, n_hist, n_cls, ssum_ref, x_ref, tpad_ref, w1_ref,
              b1_ref, w2_ref, b2_ref, out_ref):
    npad = jnp.sum((x_ref[...] == pad_idx).astype(jnp.float32), axis=1,
                   keepdims=True)
    cnt = jnp.maximum(jnp.float32(n_hist) - npad, 1.0)
    pooled = (ssum_ref[...] - npad * tpad_ref[...]) / cnt
    h = jnp.maximum(
        jnp.dot(pooled, w1_ref[...], preferred_element_type=jnp.float32)
        + b1_ref[...], 0.0)
    logits = (jnp.dot(h, w2_ref[...], preferred_element_type=jnp.float32)
              + b2_ref[...])
    col = lax.broadcasted_iota(jnp.int32, logits.shape, 1)
    valid = col < n_cls
    z = jnp.where(valid, logits, -1e30)
    m = jnp.max(z, axis=1, keepdims=True)
    e = jnp.exp(z - m) * valid.astype(jnp.float32)
    out_ref[...] = e / jnp.sum(e, axis=1, keepdims=True)


def kernel(x, table, W1, b1, W2, b2):
    B, L = x.shape
    V, D = table.shape
    pad_idx = V - 1
    n_cls = W2.shape[1]

    ssum = _sc_pool_sum(x, table)

    lane = 128
    w2p = jnp.zeros((W2.shape[0], lane), W2.dtype).at[:, :n_cls].set(W2)
    b2p = jnp.zeros((1, lane), b2.dtype).at[0, :n_cls].set(b2)
    tpad = table[pad_idx][None, :]

    blk = 256
    body = functools.partial(_mlp_body, pad_idx, L, n_cls)
    out = pl.pallas_call(
        body,
        grid=(B // blk,),
        in_specs=[
            pl.BlockSpec((blk, D), lambda i: (i, 0)),
            pl.BlockSpec((blk, L), lambda i: (i, 0)),
            pl.BlockSpec((1, D), lambda i: (0, 0)),
            pl.BlockSpec((D, W1.shape[1]), lambda i: (0, 0)),
            pl.BlockSpec((1, W1.shape[1]), lambda i: (0, 0)),
            pl.BlockSpec((W1.shape[1], lane), lambda i: (0, 0)),
            pl.BlockSpec((1, lane), lambda i: (0, 0)),
        ],
        out_specs=pl.BlockSpec((blk, lane), lambda i: (i, 0)),
        out_shape=jax.ShapeDtypeStruct((B, lane), jnp.float32),
    )(ssum, x, tpad, W1, b1[None, :], w2p, b2p)

    return out[:, :n_cls]


# speedup vs baseline: 1.1433x; 1.1433x over previous
import functools

import jax
import jax.numpy as jnp
from jax import lax
from jax.experimental import pallas as pl
from jax.experimental.pallas import tpu as pltpu
from jax.experimental.pallas import tpu_sc as plsc

NC = 2
NS = 16
NW = NC * NS
LANES = 16


def _sc_pool_sum(x, table):
    B, L = x.shape
    V, D = table.shape
    assert B % NW == 0 and D % LANES == 0
    rpw = B // NW
    nch = D // LANES
    rpg = 2
    ng = rpw // rpg
    nbuf = 4

    x2 = x.reshape(B // rpg, rpg * L)

    mesh = plsc.VectorSubcoreMesh(core_axis_name="c", subcore_axis_name="s")

    @functools.partial(
        pl.kernel,
        out_type=jax.ShapeDtypeStruct((B, D), jnp.float32),
        mesh=mesh,
        scratch_types=[
            pltpu.VMEM((ng, rpg * L), jnp.int32),
            [pltpu.VMEM((rpg * L, D), jnp.float32) for _ in range(nbuf)],
            pltpu.VMEM((rpw, D), jnp.float32),
            [pltpu.SemaphoreType.DMA for _ in range(nbuf)],
        ],
    )
    def pool(x_hbm, table_hbm, out_hbm, idx_v, bufs, out_v, sems):
        wid = lax.axis_index("s") * NC + lax.axis_index("c")
        pltpu.sync_copy(x_hbm.at[pl.ds(wid * ng, ng)], idx_v)

        for g in range(nbuf - 1):
            pltpu.async_copy(table_hbm.at[idx_v.at[g]], bufs[g], sems[g])

        def body(i, carry):
            for p in range(nbuf):
                g = i * nbuf + p
                buf = bufs[p]
                pltpu.make_async_copy(table_hbm.at[idx_v.at[0]], buf,
                                      sems[p]).wait()

                nxt = g + nbuf - 1
                pn = (p + nbuf - 1) % nbuf

                @pl.when(nxt < ng)
                def _():
                    pltpu.async_copy(table_hbm.at[idx_v.at[nxt]],
                                     bufs[pn], sems[pn])

                for half in range(rpg):
                    r = g * rpg + half
                    rb = half * L
                    for c in range(nch):
                        ch = pl.ds(c * LANES, LANES)
                        acc = buf[rb, ch] + buf[rb + 1, ch]
                        for j in range(2, L - 1, 2):
                            acc = acc + (buf[rb + j, ch] + buf[rb + j + 1, ch])
                        if L % 2:
                            acc = acc + buf[rb + L - 1, ch]
                        out_v[r, ch] = acc
            return carry

        lax.fori_loop(0, ng // nbuf, body, 0)
        pltpu.sync_copy(out_v, out_hbm.at[pl.ds(wid * rpw, rpw)])

    return pool(x2, table)


def _mlp_body(pad_idx, n_hist, n_cls, ssum_ref, x_ref, tpad_ref, w1_ref,
              b1_ref, w2_ref, b2_ref, out_ref):
    npad = jnp.sum((x_ref[...] == pad_idx).astype(jnp.float32), axis=1,
                   keepdims=True)
    cnt = jnp.maximum(jnp.float32(n_hist) - npad, 1.0)
    pooled = (ssum_ref[...] - npad * tpad_ref[...]) / cnt
    h = jnp.maximum(
        jnp.dot(pooled, w1_ref[...], preferred_element_type=jnp.float32)
        + b1_ref[...], 0.0)
    logits = (jnp.dot(h, w2_ref[...], preferred_element_type=jnp.float32)
              + b2_ref[...])
    col = lax.broadcasted_iota(jnp.int32, logits.shape, 1)
    valid = col < n_cls
    z = jnp.where(valid, logits, -1e30)
    m = jnp.max(z, axis=1, keepdims=True)
    e = jnp.exp(z - m) * valid.astype(jnp.float32)
    out_ref[...] = e / jnp.sum(e, axis=1, keepdims=True)


def kernel(x, table, W1, b1, W2, b2):
    B, L = x.shape
    V, D = table.shape
    pad_idx = V - 1
    n_cls = W2.shape[1]

    ssum = _sc_pool_sum(x, table)

    lane = 128
    w2p = jnp.zeros((W2.shape[0], lane), W2.dtype).at[:, :n_cls].set(W2)
    b2p = jnp.zeros((1, lane), b2.dtype).at[0, :n_cls].set(b2)
    tpad = table[pad_idx][None, :]

    blk = 256
    body = functools.partial(_mlp_body, pad_idx, L, n_cls)
    out = pl.pallas_call(
        body,
        grid=(B // blk,),
        in_specs=[
            pl.BlockSpec((blk, D), lambda i: (i, 0)),
            pl.BlockSpec((blk, L), lambda i: (i, 0)),
            pl.BlockSpec((1, D), lambda i: (0, 0)),
            pl.BlockSpec((D, W1.shape[1]), lambda i: (0, 0)),
            pl.BlockSpec((1, W1.shape[1]), lambda i: (0, 0)),
            pl.BlockSpec((W1.shape[1], lane), lambda i: (0, 0)),
            pl.BlockSpec((1, lane), lambda i: (0, 0)),
        ],
        out_specs=pl.BlockSpec((blk, lane), lambda i: (i, 0)),
        out_shape=jax.ShapeDtypeStruct((B, lane), jnp.float32),
    )(ssum, x, tpad, W1, b1[None, :], w2p, b2p)

    return out[:, :n_cls]
